# Initial kernel scaffold; baseline (speedup 1.0000x reference)
import jax
import jax.numpy as jnp
from jax.experimental import pallas as pl


def kernel(encoded_latents, embedding):
    raise NotImplementedError("write your pallas kernel here")


# speedup vs baseline: 1.3641x; 1.3641x over previous
import jax
import jax.numpy as jnp
from jax import lax
from jax.experimental import pallas as pl

_COMMITMENT_WEIGHT = 0.25
_BLK = 256


def _vq_body(z_ref, e_ref, zsq_ref, esq_ref, idx_ref, msum_ref):
    z = z_ref[...].astype(jnp.bfloat16).astype(jnp.float32)
    e = e_ref[...].astype(jnp.bfloat16).astype(jnp.float32)
    ze2 = 2.0 * lax.dot_general(
        z, e, (((1,), (1,)), ((), ())), preferred_element_type=jnp.float32)
    d = (zsq_ref[...] - ze2) + esq_ref[...]
    m = jnp.min(d, axis=1, keepdims=True)
    iota = lax.broadcasted_iota(jnp.int32, d.shape, 1)
    idx = jnp.min(jnp.where(d <= m, iota, jnp.int32(2 ** 30)),
                  axis=1, keepdims=True)
    idx_ref[...] = idx
    msum_ref[...] = jnp.sum(m).reshape(1, 1, 1)


def kernel(encoded_latents, embedding):
    encoded_latents = encoded_latents.astype(jnp.float32)
    B, N, D = encoded_latents.shape
    CB = embedding.shape[0]
    rows = B * N
    grid = rows // _BLK
    z = encoded_latents.reshape(rows, D)
    zsq = jnp.sum(z ** 2, axis=1, keepdims=True)
    esq = jnp.sum(embedding ** 2, axis=1)[None, :]

    idx2d, msum = pl.pallas_call(
        _vq_body,
        grid=(grid,),
        in_specs=[
            pl.BlockSpec((_BLK, D), lambda i: (i, 0)),
            pl.BlockSpec((CB, D), lambda i: (0, 0)),
            pl.BlockSpec((_BLK, 1), lambda i: (i, 0)),
            pl.BlockSpec((1, CB), lambda i: (0, 0)),
        ],
        out_specs=[
            pl.BlockSpec((_BLK, 1), lambda i: (i, 0)),
            pl.BlockSpec((1, 1, 1), lambda i: (i, 0, 0)),
        ],
        out_shape=[
            jax.ShapeDtypeStruct((rows, 1), jnp.int32),
            jax.ShapeDtypeStruct((grid, 1, 1), jnp.float32),
        ],
    )(z, embedding, zsq, esq)

    quantized_indices = idx2d.reshape(B, N)
    commitment_loss = _COMMITMENT_WEIGHT * (jnp.sum(msum) / (rows * D))
    return (encoded_latents, quantized_indices, commitment_loss)
